# baseline (device time: 23660 ns/iter reference)
import jax
import jax.numpy as jnp
from jax import lax
from jax.experimental import pallas as pl
from jax.experimental.pallas import tpu as pltpu

K = 8


def kernel(x):
    m, n = x.shape
    half = m // 2
    ck = half // K

    def body(x_ref, out_ref, vxs, vxo, vbs, vbo,
             ins_s, ins_o, outs_s, outs_o,
             ysend_sems, yrecv_sems, xsend_sems, xrecv_sems):
        my_x = lax.axis_index("x")
        my_y = lax.axis_index("y")
        my_z = lax.axis_index("z")
        y_nbr = (my_x, 1 - my_y, my_z)
        x_nbr = (1 - my_x, my_y, my_z)

        barrier = pltpu.get_barrier_semaphore()
        for nbr in (y_nbr, x_nbr):
            pl.semaphore_signal(
                barrier, inc=1, device_id=nbr,
                device_id_type=pl.DeviceIdType.MESH,
            )

        mine = my_y * m
        other = (1 - my_y) * m
        x_half = my_x * half
        x_other = (1 - my_x) * half

        in_s = []
        for j in range(K):
            c = pltpu.make_async_copy(
                x_ref.at[pl.ds(x_half + j * ck, ck)],
                vxs.at[pl.ds(j * ck, ck)],
                ins_s.at[j],
            )
            c.start()
            in_s.append(c)
        in_o = []
        for j in range(K):
            c = pltpu.make_async_copy(
                x_ref.at[pl.ds(x_other + j * ck, ck)],
                vxo.at[pl.ds(j * ck, ck)],
                ins_o.at[j],
            )
            c.start()
            in_o.append(c)

        pl.semaphore_wait(barrier, 2)

        y_rdmas = []
        out_s = []
        for j in range(K):
            in_s[j].wait()
            rows = pl.ds(j * ck, ck)
            vbs[rows, :] = vxs[rows, :].astype(jnp.bfloat16)
            grows = pl.ds(mine + x_half + j * ck, ck)
            r = pltpu.make_async_remote_copy(
                src_ref=vbs.at[rows],
                dst_ref=out_ref.at[grows],
                send_sem=ysend_sems.at[j],
                recv_sem=yrecv_sems.at[j],
                device_id=y_nbr,
                device_id_type=pl.DeviceIdType.MESH,
            )
            r.start()
            y_rdmas.append(r)
            c = pltpu.make_async_copy(vbs.at[rows], out_ref.at[grows], outs_s.at[j])
            c.start()
            out_s.append(c)

        out_o = []
        for j in range(K):
            in_o[j].wait()
            rows = pl.ds(j * ck, ck)
            vbo[rows, :] = vxo[rows, :].astype(jnp.bfloat16)
            c = pltpu.make_async_copy(
                vbo.at[rows],
                out_ref.at[pl.ds(mine + x_other + j * ck, ck)],
                outs_o.at[j],
            )
            c.start()
            out_o.append(c)

        x_rdmas = []
        for j in range(K):
            y_rdmas[j].wait_recv()
            grows = pl.ds(other + x_half + j * ck, ck)
            r = pltpu.make_async_remote_copy(
                src_ref=out_ref.at[grows],
                dst_ref=out_ref.at[grows],
                send_sem=xsend_sems.at[j],
                recv_sem=xrecv_sems.at[j],
                device_id=x_nbr,
                device_id_type=pl.DeviceIdType.MESH,
            )
            r.start()
            x_rdmas.append(r)

        for j in range(K):
            x_rdmas[j].wait_recv()
        for j in range(K):
            y_rdmas[j].wait_send()
            x_rdmas[j].wait_send()
            out_s[j].wait()
            out_o[j].wait()

    return pl.pallas_call(
        body,
        out_shape=jax.ShapeDtypeStruct((2 * m, n), jnp.bfloat16),
        in_specs=[pl.BlockSpec(memory_space=pl.ANY)],
        out_specs=pl.BlockSpec(memory_space=pl.ANY),
        scratch_shapes=[
            pltpu.VMEM((half, n), x.dtype),
            pltpu.VMEM((half, n), x.dtype),
            pltpu.VMEM((half, n), jnp.bfloat16),
            pltpu.VMEM((half, n), jnp.bfloat16),
            pltpu.SemaphoreType.DMA((K,)),
            pltpu.SemaphoreType.DMA((K,)),
            pltpu.SemaphoreType.DMA((K,)),
            pltpu.SemaphoreType.DMA((K,)),
            pltpu.SemaphoreType.DMA((K,)),
            pltpu.SemaphoreType.DMA((K,)),
            pltpu.SemaphoreType.DMA((K,)),
            pltpu.SemaphoreType.DMA((K,)),
        ],
        compiler_params=pltpu.CompilerParams(collective_id=0),
    )(x)


# device time: 21182 ns/iter; 1.1170x vs baseline; 1.1170x over previous
import jax
import jax.numpy as jnp
from jax import lax
from jax.experimental import pallas as pl
from jax.experimental.pallas import tpu as pltpu

CK = 128


def kernel(x):
    m, n = x.shape
    q = m // 4
    KQ = q // CK
    KB = KQ // 2

    def body(x_ref, out_ref, ys, yr, xsA, xrA, zsA, zrA, xsB, xrB, zsB, zrB):
        my_x = lax.axis_index("x")
        my_y = lax.axis_index("y")
        my_z = lax.axis_index("z")
        qz = lax.rem(my_z, 2)
        y_nbr = (my_x, 1 - my_y, my_z)
        x_nbr = (1 - my_x, my_y, my_z)
        z_nbr = (my_x, my_y, my_z + 1 - 2 * qz)

        barrier = pltpu.get_barrier_semaphore()
        for nbr in (y_nbr, x_nbr, z_nbr):
            pl.semaphore_signal(
                barrier, inc=1, device_id=nbr,
                device_id_type=pl.DeviceIdType.MESH,
            )
        pl.semaphore_wait(barrier, 3)

        mine = my_y * m
        other = (1 - my_y) * m
        Qd = my_x * 2 * q + qz * q
        Qx = (1 - my_x) * 2 * q + qz * q
        Qz = my_x * 2 * q + (1 - qz) * q
        Qg = (1 - my_x) * 2 * q + (1 - qz) * q

        def rc(off, ssem, rsem, dev):
            return pltpu.make_async_remote_copy(
                src_ref=out_ref.at[pl.ds(off, CK)],
                dst_ref=out_ref.at[pl.ds(off, CK)],
                send_sem=ssem,
                recv_sem=rsem,
                device_id=dev,
                device_id_type=pl.DeviceIdType.MESH,
            )

        def rwait(rsem):
            d = pltpu.make_async_remote_copy(
                src_ref=out_ref.at[pl.ds(other, CK)],
                dst_ref=out_ref.at[pl.ds(other, CK)],
                send_sem=ys.at[0],
                recv_sem=rsem,
                device_id=y_nbr,
                device_id_type=pl.DeviceIdType.MESH,
            )
            d.wait_recv()

        y_rd = []
        for j in range(KQ):
            rows = pl.ds(Qd + j * CK, CK)
            out_ref[pl.ds(mine + Qd + j * CK, CK), :] = x_ref[rows, :].astype(
                jnp.bfloat16
            )
            r = rc(mine + Qd + j * CK, ys.at[j], yr.at[j], y_nbr)
            r.start()
            y_rd.append(r)

        for off in (Qx, Qz, Qg):
            out_ref[pl.ds(mine + off, q), :] = x_ref[pl.ds(off, q), :].astype(
                jnp.bfloat16
            )

        xA, zA = [], []
        for j in range(KQ):
            y_rd[j].wait_recv()
            o = other + Qd + j * CK
            r1 = rc(o, xsA.at[j], xrA.at[j], x_nbr)
            r1.start()
            xA.append(r1)
            r2 = rc(o, zsA.at[j], zrA.at[j], z_nbr)
            r2.start()
            zA.append(r2)

        xB = []
        for i in range(KB):
            rwait(zrA.at[i])
            r = rc(other + Qz + i * CK, xsB.at[i], xrB.at[i], x_nbr)
            r.start()
            xB.append(r)
        zB = []
        for i in range(KB):
            rwait(xrA.at[KB + i])
            r = rc(other + Qx + (KB + i) * CK, zsB.at[i], zrB.at[i], z_nbr)
            r.start()
            zB.append(r)

        for i in range(KB):
            rwait(xrA.at[i])
        for i in range(KB):
            rwait(zrA.at[KB + i])
        for i in range(KB):
            rwait(xrB.at[i])
        for i in range(KB):
            rwait(zrB.at[i])

        for j in range(KQ):
            y_rd[j].wait_send()
            xA[j].wait_send()
            zA[j].wait_send()
        for i in range(KB):
            xB[i].wait_send()
            zB[i].wait_send()

    return pl.pallas_call(
        body,
        out_shape=jax.ShapeDtypeStruct((2 * m, n), jnp.bfloat16),
        in_specs=[pl.BlockSpec(memory_space=pltpu.VMEM)],
        out_specs=pl.BlockSpec(memory_space=pltpu.VMEM),
        scratch_shapes=[
            pltpu.SemaphoreType.DMA((KQ,)),
            pltpu.SemaphoreType.DMA((KQ,)),
            pltpu.SemaphoreType.DMA((KQ,)),
            pltpu.SemaphoreType.DMA((KQ,)),
            pltpu.SemaphoreType.DMA((KQ,)),
            pltpu.SemaphoreType.DMA((KQ,)),
            pltpu.SemaphoreType.DMA((KB,)),
            pltpu.SemaphoreType.DMA((KB,)),
            pltpu.SemaphoreType.DMA((KB,)),
            pltpu.SemaphoreType.DMA((KB,)),
        ],
        compiler_params=pltpu.CompilerParams(collective_id=0),
    )(x)
